# baseline (speedup 1.0000x reference)
import functools

import jax
import jax.numpy as jnp
from jax import lax
from jax.experimental import pallas as pl
from jax.experimental.pallas import tpu as pltpu
from jax.experimental.pallas import tpu_sc as plsc

B = 4
NG = 2500
EG = 80000
N = B * NG
E = B * EG
H = 128
HH = H // 2
L = 3
ES = 50
ESP = 64
CUTOFF = 5.0
STEP = 0.1
NUM_EMB = 119
LOG2 = 0.6931471805599453
SCALE_EPS = 1e-6

NC = 2
NS = 16
EPT = E // NS
CH = 80
NCHUNK = EPT // CH
NBUF = 4


def _ssp(x):
    return jnp.maximum(x, 0.0) + jnp.log(1.0 + jnp.exp(-jnp.abs(x))) - LOG2


def _sp(x):
    return jnp.maximum(x, 0.0) + jnp.log(1.0 + jnp.exp(-jnp.abs(x)))


def _embed_body(nodes_ref, emb_ref, out_ref):
    ids = nodes_ref[...]
    lane = lax.broadcasted_iota(jnp.int32, (1000, H), 1)
    oh = (lane == ids).astype(jnp.float32)
    out_ref[...] = jnp.dot(oh, emb_ref[...],
                           preferred_element_type=jnp.float32)


def _tc_embed(nodes2d, emb_pad):
    return pl.pallas_call(
        _embed_body,
        grid=(N // 1000,),
        in_specs=[
            pl.BlockSpec((1000, 1), lambda i: (i, 0)),
            pl.BlockSpec((H, H), lambda i: (0, 0)),
        ],
        out_specs=pl.BlockSpec((1000, H), lambda i: (i, 0)),
        out_shape=jax.ShapeDtypeStruct((N, H), jnp.float32),
    )(nodes2d, emb_pad)


def _gates_body(ef_ref, w1_ref, b1_ref, w2_ref, b2_ref, out_ref):
    x = ef_ref[...]
    k = lax.broadcasted_iota(jnp.int32, (512, ESP), 1).astype(jnp.float32) * STEP
    es = jnp.exp(-((x - k) ** 2) * (1.0 / (2.0 * STEP * STEP)))
    h1 = lax.dot_general(es, w1_ref[...], (((1,), (1,)), ((), ())),
                         preferred_element_type=jnp.float32) + b1_ref[...]
    y = _ssp(h1)
    w2 = w2_ref[...]
    b2 = b2_ref[...]
    out_ref[0] = lax.dot_general(y, w2[0:HH, :], (((1,), (1,)), ((), ())),
                                 preferred_element_type=jnp.float32) + b2[:, 0:HH]
    out_ref[1] = lax.dot_general(y, w2[HH:H, :], (((1,), (1,)), ((), ())),
                                 preferred_element_type=jnp.float32) + b2[:, HH:H]


def _tc_gates(ef2d, w1p, b1, w2, b2):
    return pl.pallas_call(
        _gates_body,
        grid=(E // 512,),
        in_specs=[
            pl.BlockSpec((512, 1), lambda i: (i, 0)),
            pl.BlockSpec((H, ESP), lambda i: (0, 0)),
            pl.BlockSpec((1, H), lambda i: (0, 0)),
            pl.BlockSpec((H, H), lambda i: (0, 0)),
            pl.BlockSpec((1, H), lambda i: (0, 0)),
        ],
        out_specs=pl.BlockSpec((NC, 512, HH), lambda i: (0, i, 0)),
        out_shape=jax.ShapeDtypeStruct((NC, E, HH), jnp.float32),
    )(ef2d, w1p, b1, w2, b2)


def _amlp_body(x_ref, w1_ref, b1_ref, w2_ref, b2_ref, out_ref):
    x = x_ref[...]
    h1 = lax.dot_general(x, w1_ref[...], (((1,), (1,)), ((), ())),
                         preferred_element_type=jnp.float32) + b1_ref[...]
    y = _ssp(h1)
    w2 = w2_ref[...]
    b2 = b2_ref[...]
    out_ref[0] = lax.dot_general(y, w2[0:HH, :], (((1,), (1,)), ((), ())),
                                 preferred_element_type=jnp.float32) + b2[:, 0:HH]
    out_ref[1] = lax.dot_general(y, w2[HH:H, :], (((1,), (1,)), ((), ())),
                                 preferred_element_type=jnp.float32) + b2[:, HH:H]


def _tc_amlp(x, w1, b1, w2, b2):
    return pl.pallas_call(
        _amlp_body,
        grid=(N // 1000,),
        in_specs=[
            pl.BlockSpec((1000, H), lambda i: (i, 0)),
            pl.BlockSpec((H, H), lambda i: (0, 0)),
            pl.BlockSpec((1, H), lambda i: (0, 0)),
            pl.BlockSpec((H, H), lambda i: (0, 0)),
            pl.BlockSpec((1, H), lambda i: (0, 0)),
        ],
        out_specs=pl.BlockSpec((NC, 1000, HH), lambda i: (0, i, 0)),
        out_shape=jax.ShapeDtypeStruct((NC, N, HH), jnp.float32),
    )(x, w1, b1, w2, b2)


def _update_body(p_ref, h_ref, w1_ref, b1_ref, w2_ref, b2_ref, out_ref):
    p0 = p_ref[0]
    p1 = p_ref[1]
    w1 = w1_ref[...]
    h1 = (lax.dot_general(p0, w1[:, 0:HH], (((1,), (1,)), ((), ())),
                          preferred_element_type=jnp.float32)
          + lax.dot_general(p1, w1[:, HH:H], (((1,), (1,)), ((), ())),
                            preferred_element_type=jnp.float32)
          + b1_ref[...])
    out_ref[...] = h_ref[...] + lax.dot_general(
        _ssp(h1), w2_ref[...], (((1,), (1,)), ((), ())),
        preferred_element_type=jnp.float32) + b2_ref[...]


def _tc_update(parts, h, w1, b1, w2, b2):
    return pl.pallas_call(
        _update_body,
        grid=(N // 1000,),
        in_specs=[
            pl.BlockSpec((NC, 1000, HH), lambda i: (0, i, 0)),
            pl.BlockSpec((1000, H), lambda i: (i, 0)),
            pl.BlockSpec((H, H), lambda i: (0, 0)),
            pl.BlockSpec((1, H), lambda i: (0, 0)),
            pl.BlockSpec((H, H), lambda i: (0, 0)),
            pl.BlockSpec((1, H), lambda i: (0, 0)),
        ],
        out_specs=pl.BlockSpec((1000, H), lambda i: (i, 0)),
        out_shape=jax.ShapeDtypeStruct((N, H), jnp.float32),
    )(parts, h, w1, b1, w2, b2)


def _readout_body(h_ref, w1_ref, b1_ref, w2_ref, b2_ref, out_ref):
    h = h_ref[...]
    h1 = lax.dot_general(h, w1_ref[...], (((1,), (1,)), ((), ())),
                         preferred_element_type=jnp.float32) + b1_ref[...]
    y = _ssp(h1)
    g = lax.broadcasted_iota(jnp.int32, (8, N), 0)
    n = lax.broadcasted_iota(jnp.int32, (8, N), 1) // NG
    sel = (g == n).astype(jnp.float32)
    gsum = jnp.dot(sel, y, preferred_element_type=jnp.float32)
    go = lax.dot_general(gsum, w2_ref[...], (((1,), (1,)), ((), ())),
                         preferred_element_type=jnp.float32) + float(NG) * b2_ref[...]
    col = lax.broadcasted_iota(jnp.int32, (8, H), 1)
    out_ref[...] = jnp.where(col == 1, _sp(go) + SCALE_EPS, go)


def _tc_readout(h, w1, b1, w2p, b2p):
    return pl.pallas_call(
        _readout_body,
        grid=(1,),
        in_specs=[
            pl.BlockSpec((N, H), lambda i: (0, 0)),
            pl.BlockSpec((H, H), lambda i: (0, 0)),
            pl.BlockSpec((1, H), lambda i: (0, 0)),
            pl.BlockSpec((H, H), lambda i: (0, 0)),
            pl.BlockSpec((1, H), lambda i: (0, 0)),
        ],
        out_specs=pl.BlockSpec((8, H), lambda i: (0, 0)),
        out_shape=jax.ShapeDtypeStruct((8, H), jnp.float32),
    )(h, w1, b1, w2p, b2p)


_sc_mesh = plsc.VectorSubcoreMesh(core_axis_name="c", subcore_axis_name="s")


@functools.partial(
    pl.kernel,
    out_type=jax.ShapeDtypeStruct((NC, N, HH), jnp.float32),
    mesh=_sc_mesh,
    compiler_params=pltpu.CompilerParams(use_tc_tiling_on_sc=False),
    scratch_types=[
        pltpu.VMEM((NCHUNK, CH), jnp.int32),
        [pltpu.VMEM((2, CH), jnp.int32)] * NBUF,
        [pltpu.VMEM((CH, HH), jnp.float32)] * NBUF,
        [pltpu.VMEM((CH, HH), jnp.float32)] * NBUF,
        pltpu.VMEM_SHARED((N, HH), jnp.float32),
        [pltpu.SemaphoreType.DMA] * NBUF,
        [pltpu.SemaphoreType.DMA] * NBUF,
        [pltpu.SemaphoreType.DMA] * NBUF,
    ],
)
def _sc_scatter(a_hbm, g_hbm, p_hbm, out_hbm,
                packv, ibuf, av, gv, acc, sa, sg, ss):
    c = lax.axis_index("c")
    s = lax.axis_index("s")

    pltpu.sync_copy(p_hbm.at[s], packv)

    zero16 = jnp.zeros((16,), jnp.float32)

    def _zrow(r, _):
        for k in range(HH // 16):
            av[0][r, pl.ds(k * 16, 16)] = zero16
        return 0

    lax.fori_loop(0, CH, _zrow, 0)
    base0 = s * 624
    for t in range(7):
        pltpu.sync_copy(av[0], acc.at[pl.ds(base0 + t * CH, CH)])
    pltpu.sync_copy(av[0].at[pl.ds(0, 64)], acc.at[pl.ds(base0 + 560, 64)])

    @pl.when(s == NS - 1)
    def _ztail():
        pltpu.sync_copy(av[0].at[pl.ds(0, 16)], acc.at[pl.ds(9984, 16)])

    plsc.subcore_barrier()

    goff = c * E + s * EPT
    aoff = c * N

    def _issue(cnum, b):
        for k in range(CH // 16):
            sl = pl.ds(k * 16, 16)
            p = packv[cnum, sl]
            ibuf[b][0, sl] = (p >> 16) + aoff
            ibuf[b][1, sl] = p & 0xFFFF
        pltpu.async_copy(g_hbm.at[pl.ds(goff + cnum * CH, CH)], gv[b], sg[b])
        pltpu.async_copy(a_hbm.at[ibuf[b].at[0]], av[b], sa[b])

    def _wait(sem):
        pltpu.make_async_copy(a_hbm.at[pl.ds(0, CH)], av[0], sem).wait()

    def _mult(b):
        def _mrow(r, _):
            for k in range(HH // 16):
                sl = pl.ds(k * 16, 16)
                av[b][r, sl] = av[b][r, sl] * gv[b][r, sl]
            return 0

        lax.fori_loop(0, CH, _mrow, 0)

    def _scatter(b):
        pltpu.async_copy(av[b], acc.at[ibuf[b].at[1]], ss[b], add=True)

    _issue(0, 0)
    _issue(1, 1)
    for j in (0, 1):
        _issue(j + 2, j + 2)
        _wait(sa[j]); _wait(sg[j])
        _mult(j)
        _scatter(j)

    def _steady(m, _):
        for b in range(NBUF):
            j = 4 * m + b + 2
            _wait(ss[b])
            _issue(j + 2, b)
            bj = (b + 2) % NBUF
            _wait(sa[bj]); _wait(sg[bj])
            _mult(bj)
            _scatter(bj)
        return 0

    lax.fori_loop(0, (NCHUNK - 6) // 4, _steady, 0)

    _wait(ss[0])
    _issue(NCHUNK - 2, 0)
    _wait(sa[2]); _wait(sg[2])
    _mult(2)
    _scatter(2)
    _wait(ss[1])
    _issue(NCHUNK - 1, 1)
    _wait(sa[3]); _wait(sg[3])
    _mult(3)
    _scatter(3)
    _wait(sa[0]); _wait(sg[0])
    _mult(0)
    _scatter(0)
    _wait(sa[1]); _wait(sg[1])
    _mult(1)
    pltpu.sync_copy(av[1], acc.at[ibuf[1].at[1]], add=True)
    _wait(ss[0]); _wait(ss[2]); _wait(ss[3])

    plsc.subcore_barrier()

    @pl.when(s == 0)
    def _writeback():
        pltpu.sync_copy(acc, out_hbm.at[c])


def kernel(nodes, num_nodes, edges, num_edges, edges_features, emb,
           W_me1, b_me1, W_me2, b_me2, W_mn1, b_mn1, W_mn2, b_mn2,
           W_st1, b_st1, W_st2, b_st2, W_ro1, b_ro1, W_ro2, b_ro2):
    nodes2d = nodes.reshape(N, 1)
    ef2d = edges_features.reshape(E, 1)
    off = (jnp.arange(B, dtype=jnp.int32) * NG)[:, None, None]
    ecat = (edges + off).reshape(E, 2)
    packed = (ecat[:, 0] * 65536 + ecat[:, 1]).reshape(NS, NCHUNK, CH)

    emb_pad = jnp.pad(emb, ((0, H - NUM_EMB), (0, 0)))
    w_me1_pad = jnp.pad(W_me1, ((0, 0), (0, 0), (0, ESP - ES)))
    w_ro2_pad = jnp.pad(W_ro2, ((0, H - 2), (0, 0)))
    b_ro2_pad = jnp.pad(b_ro2, ((0, H - 2),)).reshape(1, H)

    h = _tc_embed(nodes2d, emb_pad)
    gs = [
        _tc_gates(ef2d, w_me1_pad[i], b_me1[i].reshape(1, H),
                  W_me2[i], b_me2[i].reshape(1, H)).reshape(NC * E, HH)
        for i in range(L)
    ]
    for i in range(L):
        a = _tc_amlp(h, W_mn1[i], b_mn1[i].reshape(1, H),
                     W_mn2[i], b_mn2[i].reshape(1, H)).reshape(NC * N, HH)
        parts = _sc_scatter(a, gs[i], packed)
        h = _tc_update(parts, h, W_st1[i], b_st1[i].reshape(1, H),
                       W_st2[i], b_st2[i].reshape(1, H))

    go = _tc_readout(h, W_ro1, b_ro1.reshape(1, H), w_ro2_pad, b_ro2_pad)
    loc = go[0:B, 0:1]
    scale = go[0:B, 1:2]
    return (loc, scale)


# speedup vs baseline: 1.4655x; 1.4655x over previous
import functools

import jax
import jax.numpy as jnp
from jax import lax
from jax.experimental import pallas as pl
from jax.experimental.pallas import tpu as pltpu
from jax.experimental.pallas import tpu_sc as plsc

B = 4
NG = 2500
EG = 80000
N = B * NG
E = B * EG
H = 128
HH = H // 2
L = 3
ES = 50
ESP = 64
CUTOFF = 5.0
STEP = 0.1
NUM_EMB = 119
LOG2 = 0.6931471805599453
SCALE_EPS = 1e-6

NC = 2
NS = 16
EPT = E // NS
CH = 80
NCHUNK = EPT // CH
NBUF = 4


def _ssp(x):
    return jnp.maximum(x, 0.0) + jnp.log(1.0 + jnp.exp(-jnp.abs(x))) - LOG2


def _sp(x):
    return jnp.maximum(x, 0.0) + jnp.log(1.0 + jnp.exp(-jnp.abs(x)))


def _embed_body(nodes_ref, emb_ref, out_ref):
    ids = nodes_ref[...]
    lane = lax.broadcasted_iota(jnp.int32, (1000, H), 1)
    oh = (lane == ids).astype(jnp.float32)
    out_ref[...] = jnp.dot(oh, emb_ref[...],
                           preferred_element_type=jnp.float32)


def _tc_embed(nodes2d, emb_pad):
    return pl.pallas_call(
        _embed_body,
        grid=(N // 1000,),
        in_specs=[
            pl.BlockSpec((1000, 1), lambda i: (i, 0)),
            pl.BlockSpec((H, H), lambda i: (0, 0)),
        ],
        out_specs=pl.BlockSpec((1000, H), lambda i: (i, 0)),
        out_shape=jax.ShapeDtypeStruct((N, H), jnp.float32),
    )(nodes2d, emb_pad)


def _gates_body(ef_ref, w1_ref, b1_ref, w2_ref, b2_ref, out_ref):
    x = ef_ref[...]
    k = lax.broadcasted_iota(jnp.int32, (512, ESP), 1).astype(jnp.float32) * STEP
    es = jnp.exp(-((x - k) ** 2) * (1.0 / (2.0 * STEP * STEP)))
    h1 = lax.dot_general(es, w1_ref[...], (((1,), (1,)), ((), ())),
                         preferred_element_type=jnp.float32) + b1_ref[...]
    out_ref[...] = lax.dot_general(_ssp(h1), w2_ref[...],
                                   (((1,), (1,)), ((), ())),
                                   preferred_element_type=jnp.float32) + b2_ref[...]


def _tc_gates(ef2d, w1p, b1, w2, b2):
    return pl.pallas_call(
        _gates_body,
        grid=(E // 512,),
        in_specs=[
            pl.BlockSpec((512, 1), lambda i: (i, 0)),
            pl.BlockSpec((H, ESP), lambda i: (0, 0)),
            pl.BlockSpec((1, H), lambda i: (0, 0)),
            pl.BlockSpec((H, H), lambda i: (0, 0)),
            pl.BlockSpec((1, H), lambda i: (0, 0)),
        ],
        out_specs=pl.BlockSpec((512, H), lambda i: (i, 0)),
        out_shape=jax.ShapeDtypeStruct((E, H), jnp.float32),
    )(ef2d, w1p, b1, w2, b2)


def _amlp_body(x_ref, w1_ref, b1_ref, w2_ref, b2_ref, out_ref):
    x = x_ref[...]
    h1 = lax.dot_general(x, w1_ref[...], (((1,), (1,)), ((), ())),
                         preferred_element_type=jnp.float32) + b1_ref[...]
    y = _ssp(h1)
    w2 = w2_ref[...]
    b2 = b2_ref[...]
    out_ref[0] = lax.dot_general(y, w2[0:HH, :], (((1,), (1,)), ((), ())),
                                 preferred_element_type=jnp.float32) + b2[:, 0:HH]
    out_ref[1] = lax.dot_general(y, w2[HH:H, :], (((1,), (1,)), ((), ())),
                                 preferred_element_type=jnp.float32) + b2[:, HH:H]


def _tc_amlp(x, w1, b1, w2, b2):
    return pl.pallas_call(
        _amlp_body,
        grid=(N // 1000,),
        in_specs=[
            pl.BlockSpec((1000, H), lambda i: (i, 0)),
            pl.BlockSpec((H, H), lambda i: (0, 0)),
            pl.BlockSpec((1, H), lambda i: (0, 0)),
            pl.BlockSpec((H, H), lambda i: (0, 0)),
            pl.BlockSpec((1, H), lambda i: (0, 0)),
        ],
        out_specs=pl.BlockSpec((NC, 1000, HH), lambda i: (0, i, 0)),
        out_shape=jax.ShapeDtypeStruct((NC, N, HH), jnp.float32),
    )(x, w1, b1, w2, b2)


def _update_body(p_ref, h_ref, w1_ref, b1_ref, w2_ref, b2_ref, out_ref):
    p0 = p_ref[0]
    p1 = p_ref[1]
    w1 = w1_ref[...]
    h1 = (lax.dot_general(p0, w1[:, 0:HH], (((1,), (1,)), ((), ())),
                          preferred_element_type=jnp.float32)
          + lax.dot_general(p1, w1[:, HH:H], (((1,), (1,)), ((), ())),
                            preferred_element_type=jnp.float32)
          + b1_ref[...])
    out_ref[...] = h_ref[...] + lax.dot_general(
        _ssp(h1), w2_ref[...], (((1,), (1,)), ((), ())),
        preferred_element_type=jnp.float32) + b2_ref[...]


def _tc_update(parts, h, w1, b1, w2, b2):
    return pl.pallas_call(
        _update_body,
        grid=(N // 1000,),
        in_specs=[
            pl.BlockSpec((NC, 1000, HH), lambda i: (0, i, 0)),
            pl.BlockSpec((1000, H), lambda i: (i, 0)),
            pl.BlockSpec((H, H), lambda i: (0, 0)),
            pl.BlockSpec((1, H), lambda i: (0, 0)),
            pl.BlockSpec((H, H), lambda i: (0, 0)),
            pl.BlockSpec((1, H), lambda i: (0, 0)),
        ],
        out_specs=pl.BlockSpec((1000, H), lambda i: (i, 0)),
        out_shape=jax.ShapeDtypeStruct((N, H), jnp.float32),
    )(parts, h, w1, b1, w2, b2)


def _readout_body(h_ref, w1_ref, b1_ref, w2_ref, b2_ref, out_ref):
    h = h_ref[...]
    h1 = lax.dot_general(h, w1_ref[...], (((1,), (1,)), ((), ())),
                         preferred_element_type=jnp.float32) + b1_ref[...]
    y = _ssp(h1)
    g = lax.broadcasted_iota(jnp.int32, (8, N), 0)
    n = lax.broadcasted_iota(jnp.int32, (8, N), 1) // NG
    sel = (g == n).astype(jnp.float32)
    gsum = jnp.dot(sel, y, preferred_element_type=jnp.float32)
    go = lax.dot_general(gsum, w2_ref[...], (((1,), (1,)), ((), ())),
                         preferred_element_type=jnp.float32) + float(NG) * b2_ref[...]
    col = lax.broadcasted_iota(jnp.int32, (8, H), 1)
    out_ref[...] = jnp.where(col == 1, _sp(go) + SCALE_EPS, go)


def _tc_readout(h, w1, b1, w2p, b2p):
    return pl.pallas_call(
        _readout_body,
        grid=(1,),
        in_specs=[
            pl.BlockSpec((N, H), lambda i: (0, 0)),
            pl.BlockSpec((H, H), lambda i: (0, 0)),
            pl.BlockSpec((1, H), lambda i: (0, 0)),
            pl.BlockSpec((H, H), lambda i: (0, 0)),
            pl.BlockSpec((1, H), lambda i: (0, 0)),
        ],
        out_specs=pl.BlockSpec((8, H), lambda i: (0, 0)),
        out_shape=jax.ShapeDtypeStruct((8, H), jnp.float32),
    )(h, w1, b1, w2p, b2p)


_sc_mesh = plsc.VectorSubcoreMesh(core_axis_name="c", subcore_axis_name="s")


@functools.partial(
    pl.kernel,
    out_type=jax.ShapeDtypeStruct((NC, N, HH), jnp.float32),
    mesh=_sc_mesh,
    compiler_params=pltpu.CompilerParams(use_tc_tiling_on_sc=False),
    scratch_types=[
        pltpu.VMEM((NCHUNK, CH), jnp.int32),
        [pltpu.VMEM((2, CH), jnp.int32)] * NBUF,
        [pltpu.VMEM((CH, HH), jnp.float32)] * NBUF,
        [pltpu.VMEM((CH, HH), jnp.float32)] * NBUF,
        pltpu.VMEM_SHARED((N, HH), jnp.float32),
        [pltpu.SemaphoreType.DMA] * NBUF,
        [pltpu.SemaphoreType.DMA] * NBUF,
        [pltpu.SemaphoreType.DMA] * NBUF,
    ],
)
def _sc_scatter(a_hbm, g_hbm, p_hbm, out_hbm,
                packv, ibuf, av, gv, acc, sa, sg, ss):
    c = lax.axis_index("c")
    s = lax.axis_index("s")

    pltpu.sync_copy(p_hbm.at[s], packv)

    zero16 = jnp.zeros((16,), jnp.float32)

    def _zrow(r, _):
        for k in range(HH // 16):
            av[0][r, pl.ds(k * 16, 16)] = zero16
        return 0

    lax.fori_loop(0, CH, _zrow, 0)
    base0 = s * 624
    for t in range(7):
        pltpu.sync_copy(av[0], acc.at[pl.ds(base0 + t * CH, CH)])
    pltpu.sync_copy(av[0].at[pl.ds(0, 64)], acc.at[pl.ds(base0 + 560, 64)])

    @pl.when(s == NS - 1)
    def _ztail():
        pltpu.sync_copy(av[0].at[pl.ds(0, 16)], acc.at[pl.ds(9984, 16)])

    plsc.subcore_barrier()

    goff = s * EPT
    aoff = c * N

    def _issue(cnum, b):
        for k in range(CH // 16):
            sl = pl.ds(k * 16, 16)
            p = packv[cnum, sl]
            ibuf[b][0, sl] = (p >> 16) + aoff
            ibuf[b][1, sl] = p & 0xFFFF

        @pl.when(c == 0)
        def _glo():
            pltpu.async_copy(g_hbm.at[pl.ds(goff + cnum * CH, CH), pl.ds(0, HH)],
                             gv[b], sg[b])

        @pl.when(c == 1)
        def _ghi():
            pltpu.async_copy(g_hbm.at[pl.ds(goff + cnum * CH, CH), pl.ds(HH, HH)],
                             gv[b], sg[b])

        pltpu.async_copy(a_hbm.at[ibuf[b].at[0]], av[b], sa[b])

    def _wait(sem):
        pltpu.make_async_copy(a_hbm.at[pl.ds(0, CH)], av[0], sem).wait()

    def _mult(b):
        def _mrow(r, _):
            for k in range(HH // 16):
                sl = pl.ds(k * 16, 16)
                av[b][r, sl] = av[b][r, sl] * gv[b][r, sl]
            return 0

        lax.fori_loop(0, CH, _mrow, 0)

    def _scatter(b):
        pltpu.async_copy(av[b], acc.at[ibuf[b].at[1]], ss[b], add=True)

    _issue(0, 0)
    _issue(1, 1)
    for j in (0, 1):
        _issue(j + 2, j + 2)
        _wait(sa[j]); _wait(sg[j])
        _mult(j)
        _scatter(j)

    def _steady(m, _):
        for b in range(NBUF):
            j = 4 * m + b + 2
            _wait(ss[b])
            _issue(j + 2, b)
            bj = (b + 2) % NBUF
            _wait(sa[bj]); _wait(sg[bj])
            _mult(bj)
            _scatter(bj)
        return 0

    lax.fori_loop(0, (NCHUNK - 6) // 4, _steady, 0)

    _wait(ss[0])
    _issue(NCHUNK - 2, 0)
    _wait(sa[2]); _wait(sg[2])
    _mult(2)
    _scatter(2)
    _wait(ss[1])
    _issue(NCHUNK - 1, 1)
    _wait(sa[3]); _wait(sg[3])
    _mult(3)
    _scatter(3)
    _wait(sa[0]); _wait(sg[0])
    _mult(0)
    _scatter(0)
    _wait(sa[1]); _wait(sg[1])
    _mult(1)
    pltpu.sync_copy(av[1], acc.at[ibuf[1].at[1]], add=True)
    _wait(ss[0]); _wait(ss[2]); _wait(ss[3])

    plsc.subcore_barrier()

    @pl.when(s == 0)
    def _writeback():
        pltpu.sync_copy(acc, out_hbm.at[c])


def kernel(nodes, num_nodes, edges, num_edges, edges_features, emb,
           W_me1, b_me1, W_me2, b_me2, W_mn1, b_mn1, W_mn2, b_mn2,
           W_st1, b_st1, W_st2, b_st2, W_ro1, b_ro1, W_ro2, b_ro2):
    nodes2d = nodes.reshape(N, 1)
    ef2d = edges_features.reshape(E, 1)
    off = (jnp.arange(B, dtype=jnp.int32) * NG)[:, None, None]
    ecat = (edges + off).reshape(E, 2)
    packed = (ecat[:, 0] * 65536 + ecat[:, 1]).reshape(NS, NCHUNK, CH)

    emb_pad = jnp.pad(emb, ((0, H - NUM_EMB), (0, 0)))
    w_me1_pad = jnp.pad(W_me1, ((0, 0), (0, 0), (0, ESP - ES)))
    w_ro2_pad = jnp.pad(W_ro2, ((0, H - 2), (0, 0)))
    b_ro2_pad = jnp.pad(b_ro2, ((0, H - 2),)).reshape(1, H)

    h = _tc_embed(nodes2d, emb_pad)
    gs = [
        _tc_gates(ef2d, w_me1_pad[i], b_me1[i].reshape(1, H),
                  W_me2[i], b_me2[i].reshape(1, H))
        for i in range(L)
    ]
    for i in range(L):
        a = _tc_amlp(h, W_mn1[i], b_mn1[i].reshape(1, H),
                     W_mn2[i], b_mn2[i].reshape(1, H)).reshape(NC * N, HH)
        parts = _sc_scatter(a, gs[i], packed)
        h = _tc_update(parts, h, W_st1[i], b_st1[i].reshape(1, H),
                       W_st2[i], b_st2[i].reshape(1, H))

    go = _tc_readout(h, W_ro1, b_ro1.reshape(1, H), w_ro2_pad, b_ro2_pad)
    loc = go[0:B, 0:1]
    scale = go[0:B, 1:2]
    return (loc, scale)


# speedup vs baseline: 2.4974x; 1.7041x over previous
import functools

import jax
import jax.numpy as jnp
from jax import lax
from jax.experimental import pallas as pl
from jax.experimental.pallas import tpu as pltpu
from jax.experimental.pallas import tpu_sc as plsc

B = 4
NG = 2500
EG = 80000
N = B * NG
E = B * EG
H = 128
HH = H // 2
L = 3
ES = 50
ESP = 64
CUTOFF = 5.0
STEP = 0.1
NUM_EMB = 119
LOG2 = 0.6931471805599453
SCALE_EPS = 1e-6

NC = 2
NS = 16
EPT = E // NS
CH = 80
NCHUNK = EPT // CH
NBUF = 4


def _ssp(x):
    return jnp.maximum(x, 0.0) + jnp.log(1.0 + jnp.exp(-jnp.abs(x))) - LOG2


def _sp(x):
    return jnp.maximum(x, 0.0) + jnp.log(1.0 + jnp.exp(-jnp.abs(x)))


def _amlp(h, w1, b1, w2, b2):
    h1 = lax.dot_general(h, w1, (((1,), (1,)), ((), ())),
                         preferred_element_type=jnp.float32) + b1
    y = _ssp(h1)
    a0 = lax.dot_general(y, w2[0:HH, :], (((1,), (1,)), ((), ())),
                         preferred_element_type=jnp.float32) + b2[:, 0:HH]
    a1 = lax.dot_general(y, w2[HH:H, :], (((1,), (1,)), ((), ())),
                         preferred_element_type=jnp.float32) + b2[:, HH:H]
    return a0, a1


def _embed_amlp_body(nodes_ref, emb_ref, w1_ref, b1_ref, w2_ref, b2_ref,
                     h_ref, a_ref):
    ids = nodes_ref[...]
    lane = lax.broadcasted_iota(jnp.int32, (1000, H), 1)
    oh = (lane == ids).astype(jnp.float32)
    h = jnp.dot(oh, emb_ref[...], preferred_element_type=jnp.float32)
    h_ref[...] = h
    a_ref[0], a_ref[1] = _amlp(h, w1_ref[...], b1_ref[...],
                               w2_ref[...], b2_ref[...])


def _tc_embed_amlp(nodes2d, emb_pad, w1, b1, w2, b2):
    return pl.pallas_call(
        _embed_amlp_body,
        grid=(N // 1000,),
        in_specs=[
            pl.BlockSpec((1000, 1), lambda i: (i, 0)),
            pl.BlockSpec((H, H), lambda i: (0, 0)),
            pl.BlockSpec((H, H), lambda i: (0, 0)),
            pl.BlockSpec((1, H), lambda i: (0, 0)),
            pl.BlockSpec((H, H), lambda i: (0, 0)),
            pl.BlockSpec((1, H), lambda i: (0, 0)),
        ],
        out_specs=[
            pl.BlockSpec((1000, H), lambda i: (i, 0)),
            pl.BlockSpec((NC, 1000, HH), lambda i: (0, i, 0)),
        ],
        out_shape=[
            jax.ShapeDtypeStruct((N, H), jnp.float32),
            jax.ShapeDtypeStruct((NC, N, HH), jnp.float32),
        ],
    )(nodes2d, emb_pad, w1, b1, w2, b2)


def _gates3_body(ef_ref, w1_ref, b1_ref, w2_ref, b2_ref,
                 o0_ref, o1_ref, o2_ref):
    x = ef_ref[...].reshape(512, 1)
    k = lax.broadcasted_iota(jnp.int32, (512, ESP), 1).astype(jnp.float32) * STEP
    es = jnp.exp(-((x - k) ** 2) * (1.0 / (2.0 * STEP * STEP)))
    for i, o_ref in enumerate((o0_ref, o1_ref, o2_ref)):
        h1 = lax.dot_general(es, w1_ref[i], (((1,), (1,)), ((), ())),
                             preferred_element_type=jnp.float32) + b1_ref[i]
        o_ref[...] = lax.dot_general(_ssp(h1), w2_ref[i],
                                     (((1,), (1,)), ((), ())),
                                     preferred_element_type=jnp.float32) + b2_ref[i]


def _tc_gates3(ef_r, w1p, b1, w2, b2):
    return pl.pallas_call(
        _gates3_body,
        grid=(E // 512,),
        in_specs=[
            pl.BlockSpec((512,), lambda i: (i,)),
            pl.BlockSpec((L, H, ESP), lambda i: (0, 0, 0)),
            pl.BlockSpec((L, 1, H), lambda i: (0, 0, 0)),
            pl.BlockSpec((L, H, H), lambda i: (0, 0, 0)),
            pl.BlockSpec((L, 1, H), lambda i: (0, 0, 0)),
        ],
        out_specs=[pl.BlockSpec((512, H), lambda i: (i, 0))] * L,
        out_shape=[jax.ShapeDtypeStruct((E, H), jnp.float32)] * L,
    )(ef_r, w1p, b1, w2, b2)


def _update_core(p_ref, h_ref, w1_ref, b1_ref, w2_ref, b2_ref):
    p0 = p_ref[0]
    p1 = p_ref[1]
    w1 = w1_ref[...]
    h1 = (lax.dot_general(p0, w1[:, 0:HH], (((1,), (1,)), ((), ())),
                          preferred_element_type=jnp.float32)
          + lax.dot_general(p1, w1[:, HH:H], (((1,), (1,)), ((), ())),
                            preferred_element_type=jnp.float32)
          + b1_ref[...])
    return h_ref[...] + lax.dot_general(
        _ssp(h1), w2_ref[...], (((1,), (1,)), ((), ())),
        preferred_element_type=jnp.float32) + b2_ref[...]


def _update_body(p_ref, h_ref, w1_ref, b1_ref, w2_ref, b2_ref, out_ref):
    out_ref[...] = _update_core(p_ref, h_ref, w1_ref, b1_ref, w2_ref, b2_ref)


def _tc_update(parts, h, w1, b1, w2, b2):
    return pl.pallas_call(
        _update_body,
        grid=(N // 1000,),
        in_specs=[
            pl.BlockSpec((NC, 1000, HH), lambda i: (0, i, 0)),
            pl.BlockSpec((1000, H), lambda i: (i, 0)),
            pl.BlockSpec((H, H), lambda i: (0, 0)),
            pl.BlockSpec((1, H), lambda i: (0, 0)),
            pl.BlockSpec((H, H), lambda i: (0, 0)),
            pl.BlockSpec((1, H), lambda i: (0, 0)),
        ],
        out_specs=pl.BlockSpec((1000, H), lambda i: (i, 0)),
        out_shape=jax.ShapeDtypeStruct((N, H), jnp.float32),
    )(parts, h, w1, b1, w2, b2)


def _update_amlp_body(p_ref, h_ref, w1_ref, b1_ref, w2_ref, b2_ref,
                      wn1_ref, bn1_ref, wn2_ref, bn2_ref, hn_ref, an_ref):
    hn = _update_core(p_ref, h_ref, w1_ref, b1_ref, w2_ref, b2_ref)
    hn_ref[...] = hn
    an_ref[0], an_ref[1] = _amlp(hn, wn1_ref[...], bn1_ref[...],
                                 wn2_ref[...], bn2_ref[...])


def _tc_update_amlp(parts, h, w1, b1, w2, b2, wn1, bn1, wn2, bn2):
    return pl.pallas_call(
        _update_amlp_body,
        grid=(N // 1000,),
        in_specs=[
            pl.BlockSpec((NC, 1000, HH), lambda i: (0, i, 0)),
            pl.BlockSpec((1000, H), lambda i: (i, 0)),
            pl.BlockSpec((H, H), lambda i: (0, 0)),
            pl.BlockSpec((1, H), lambda i: (0, 0)),
            pl.BlockSpec((H, H), lambda i: (0, 0)),
            pl.BlockSpec((1, H), lambda i: (0, 0)),
            pl.BlockSpec((H, H), lambda i: (0, 0)),
            pl.BlockSpec((1, H), lambda i: (0, 0)),
            pl.BlockSpec((H, H), lambda i: (0, 0)),
            pl.BlockSpec((1, H), lambda i: (0, 0)),
        ],
        out_specs=[
            pl.BlockSpec((1000, H), lambda i: (i, 0)),
            pl.BlockSpec((NC, 1000, HH), lambda i: (0, i, 0)),
        ],
        out_shape=[
            jax.ShapeDtypeStruct((N, H), jnp.float32),
            jax.ShapeDtypeStruct((NC, N, HH), jnp.float32),
        ],
    )(parts, h, w1, b1, w2, b2, wn1, bn1, wn2, bn2)


def _readout_body(h_ref, w1_ref, b1_ref, w2_ref, b2_ref, out_ref):
    h = h_ref[...]
    h1 = lax.dot_general(h, w1_ref[...], (((1,), (1,)), ((), ())),
                         preferred_element_type=jnp.float32) + b1_ref[...]
    y = _ssp(h1)
    g = lax.broadcasted_iota(jnp.int32, (8, N), 0)
    n = lax.broadcasted_iota(jnp.int32, (8, N), 1) // NG
    sel = (g == n).astype(jnp.float32)
    gsum = jnp.dot(sel, y, preferred_element_type=jnp.float32)
    go = lax.dot_general(gsum, w2_ref[...], (((1,), (1,)), ((), ())),
                         preferred_element_type=jnp.float32) + float(NG) * b2_ref[...]
    col = lax.broadcasted_iota(jnp.int32, (8, H), 1)
    out_ref[...] = jnp.where(col == 1, _sp(go) + SCALE_EPS, go)


def _tc_readout(h, w1, b1, w2p, b2p):
    return pl.pallas_call(
        _readout_body,
        grid=(1,),
        in_specs=[
            pl.BlockSpec((N, H), lambda i: (0, 0)),
            pl.BlockSpec((H, H), lambda i: (0, 0)),
            pl.BlockSpec((1, H), lambda i: (0, 0)),
            pl.BlockSpec((H, H), lambda i: (0, 0)),
            pl.BlockSpec((1, H), lambda i: (0, 0)),
        ],
        out_specs=pl.BlockSpec((8, H), lambda i: (0, 0)),
        out_shape=jax.ShapeDtypeStruct((8, H), jnp.float32),
    )(h, w1, b1, w2p, b2p)


_sc_mesh = plsc.VectorSubcoreMesh(core_axis_name="c", subcore_axis_name="s")


@functools.partial(
    pl.kernel,
    out_type=jax.ShapeDtypeStruct((NC, N, HH), jnp.float32),
    mesh=_sc_mesh,
    compiler_params=pltpu.CompilerParams(use_tc_tiling_on_sc=False),
    scratch_types=[
        pltpu.VMEM((NCHUNK, CH), jnp.int32),
        [pltpu.VMEM((2, CH), jnp.int32)] * NBUF,
        [pltpu.VMEM((CH, HH), jnp.float32)] * NBUF,
        [pltpu.VMEM((CH, HH), jnp.float32)] * NBUF,
        pltpu.VMEM_SHARED((N, HH), jnp.float32),
        [pltpu.SemaphoreType.DMA] * NBUF,
        [pltpu.SemaphoreType.DMA] * NBUF,
        [pltpu.SemaphoreType.DMA] * NBUF,
    ],
)
def _sc_scatter(a_hbm, g_hbm, p_hbm, out_hbm,
                packv, ibuf, av, gv, acc, sa, sg, ss):
    c = lax.axis_index("c")
    s = lax.axis_index("s")

    pltpu.sync_copy(p_hbm.at[s], packv)

    zero16 = jnp.zeros((16,), jnp.float32)

    def _zrow(r, _):
        for k in range(HH // 16):
            av[0][r, pl.ds(k * 16, 16)] = zero16
        return 0

    lax.fori_loop(0, CH, _zrow, 0)
    base0 = s * 624
    for t in range(7):
        pltpu.sync_copy(av[0], acc.at[pl.ds(base0 + t * CH, CH)])
    pltpu.sync_copy(av[0].at[pl.ds(0, 64)], acc.at[pl.ds(base0 + 560, 64)])

    @pl.when(s == NS - 1)
    def _ztail():
        pltpu.sync_copy(av[0].at[pl.ds(0, 16)], acc.at[pl.ds(9984, 16)])

    plsc.subcore_barrier()

    goff = s * EPT
    aoff = c * N

    def _issue(cnum, b):
        for k in range(CH // 16):
            sl = pl.ds(k * 16, 16)
            p = packv[cnum, sl]
            ibuf[b][0, sl] = (p >> 16) + aoff
            ibuf[b][1, sl] = p & 0xFFFF

        @pl.when(c == 0)
        def _glo():
            pltpu.async_copy(g_hbm.at[pl.ds(goff + cnum * CH, CH), pl.ds(0, HH)],
                             gv[b], sg[b])

        @pl.when(c == 1)
        def _ghi():
            pltpu.async_copy(g_hbm.at[pl.ds(goff + cnum * CH, CH), pl.ds(HH, HH)],
                             gv[b], sg[b])

        pltpu.async_copy(a_hbm.at[ibuf[b].at[0]], av[b], sa[b])

    def _wait(sem):
        pltpu.make_async_copy(a_hbm.at[pl.ds(0, CH)], av[0], sem).wait()

    def _mult(b):
        def _mrow(r, _):
            for k in range(HH // 16):
                sl = pl.ds(k * 16, 16)
                av[b][r, sl] = av[b][r, sl] * gv[b][r, sl]
            return 0

        lax.fori_loop(0, CH, _mrow, 0)

    def _scatter(b):
        pltpu.async_copy(av[b], acc.at[ibuf[b].at[1]], ss[b], add=True)

    _issue(0, 0)
    _issue(1, 1)
    for j in (0, 1):
        _issue(j + 2, j + 2)
        _wait(sa[j]); _wait(sg[j])
        _mult(j)
        _scatter(j)

    def _steady(m, _):
        for b in range(NBUF):
            j = 4 * m + b + 2
            _wait(ss[b])
            _issue(j + 2, b)
            bj = (b + 2) % NBUF
            _wait(sa[bj]); _wait(sg[bj])
            _mult(bj)
            _scatter(bj)
        return 0

    lax.fori_loop(0, (NCHUNK - 6) // 4, _steady, 0)

    _wait(ss[0])
    _issue(NCHUNK - 2, 0)
    _wait(sa[2]); _wait(sg[2])
    _mult(2)
    _scatter(2)
    _wait(ss[1])
    _issue(NCHUNK - 1, 1)
    _wait(sa[3]); _wait(sg[3])
    _mult(3)
    _scatter(3)
    _wait(sa[0]); _wait(sg[0])
    _mult(0)
    _scatter(0)
    _wait(sa[1]); _wait(sg[1])
    _mult(1)
    pltpu.sync_copy(av[1], acc.at[ibuf[1].at[1]], add=True)
    _wait(ss[0]); _wait(ss[2]); _wait(ss[3])

    plsc.subcore_barrier()

    @pl.when(s == 0)
    def _writeback():
        pltpu.sync_copy(acc, out_hbm.at[c])


def kernel(nodes, num_nodes, edges, num_edges, edges_features, emb,
           W_me1, b_me1, W_me2, b_me2, W_mn1, b_mn1, W_mn2, b_mn2,
           W_st1, b_st1, W_st2, b_st2, W_ro1, b_ro1, W_ro2, b_ro2):
    nodes2d = nodes.reshape(N, 1)
    ef_r = edges_features.reshape(E)
    off = (jnp.arange(B, dtype=jnp.int32) * NG)[:, None, None]
    ecat = (edges + off).reshape(E, 2)
    packed = (ecat[:, 0] * 65536 + ecat[:, 1]).reshape(NS, NCHUNK, CH)

    emb_pad = jnp.pad(emb, ((0, H - NUM_EMB), (0, 0)))
    w_me1_pad = jnp.pad(W_me1, ((0, 0), (0, 0), (0, ESP - ES)))
    w_ro2_pad = jnp.pad(W_ro2, ((0, H - 2), (0, 0)))
    b_ro2_pad = jnp.pad(b_ro2, ((0, H - 2),)).reshape(1, H)

    gs = _tc_gates3(ef_r, w_me1_pad, b_me1.reshape(L, 1, H),
                    W_me2, b_me2.reshape(L, 1, H))
    h, a = _tc_embed_amlp(nodes2d, emb_pad, W_mn1[0], b_mn1[0].reshape(1, H),
                          W_mn2[0], b_mn2[0].reshape(1, H))
    for i in range(L):
        parts = _sc_scatter(a.reshape(NC * N, HH), gs[i], packed)
        if i < L - 1:
            h, a = _tc_update_amlp(
                parts, h, W_st1[i], b_st1[i].reshape(1, H),
                W_st2[i], b_st2[i].reshape(1, H),
                W_mn1[i + 1], b_mn1[i + 1].reshape(1, H),
                W_mn2[i + 1], b_mn2[i + 1].reshape(1, H))
        else:
            h = _tc_update(parts, h, W_st1[i], b_st1[i].reshape(1, H),
                           W_st2[i], b_st2[i].reshape(1, H))

    go = _tc_readout(h, W_ro1, b_ro1.reshape(1, H), w_ro2_pad, b_ro2_pad)
    loc = go[0:B, 0:1]
    scale = go[0:B, 1:2]
    return (loc, scale)
